# baseline (speedup 1.0000x reference)
import functools

import jax
import jax.numpy as jnp
from jax import lax
from jax.experimental import pallas as pl
from jax.experimental.pallas import tpu as pltpu
from jax.experimental.pallas import tpu_sc as plsc

NUM_CLASSES = 1000000
EMBED_DIM = 64
BATCH = 16384

NC = 2
NS = 16
NW = NC * NS
B_PER_W = BATCH // NW
CHUNK = 128
N_CHUNKS = B_PER_W // CHUNK


def _sc_gather(table, idx3):
    mesh = plsc.VectorSubcoreMesh(core_axis_name="c", subcore_axis_name="s")

    @functools.partial(
        pl.kernel,
        mesh=mesh,
        out_type=jax.ShapeDtypeStruct((BATCH, EMBED_DIM), jnp.float32),
        scratch_types=[
            pltpu.VMEM((N_CHUNKS, CHUNK), jnp.int32),
            pltpu.VMEM((N_CHUNKS, CHUNK, EMBED_DIM), jnp.float32),
            pltpu.SemaphoreType.DMA,
        ],
        compiler_params=pltpu.CompilerParams(use_tc_tiling_on_sc=False),
    )
    def gather(table_hbm, idx_hbm, out_hbm, idx_v, rows_v, sem):
        wid = lax.axis_index("s") * NC + lax.axis_index("c")
        base = wid * B_PER_W
        pltpu.sync_copy(idx_hbm.at[wid], idx_v)
        copies = []
        for j in range(N_CHUNKS):
            copies.append(
                pltpu.async_copy(table_hbm.at[idx_v.at[j]], rows_v.at[j], sem)
            )
        for c in copies:
            c.wait()
        for j in range(N_CHUNKS):
            pltpu.sync_copy(
                rows_v.at[j], out_hbm.at[pl.ds(base + j * CHUNK, CHUNK)]
            )

    return gather(table, idx3)


def _mlp_body(emb_ref, w1_ref, b1_ref, w2_ref, b2_ref, out_ref):
    x = emb_ref[...]
    h = jnp.dot(x, w1_ref[...], preferred_element_type=jnp.float32) + b1_ref[...]
    h = h * jax.nn.sigmoid(h)
    out_ref[...] = (
        jnp.dot(h, w2_ref[...], preferred_element_type=jnp.float32) + b2_ref[...]
    )


def _tc_mlp(emb, W1, b1, W2, b2):
    blk = 2048
    grid = (BATCH // blk,)
    return pl.pallas_call(
        _mlp_body,
        grid=grid,
        in_specs=[
            pl.BlockSpec((blk, EMBED_DIM), lambda i: (i, 0)),
            pl.BlockSpec((EMBED_DIM, EMBED_DIM), lambda i: (0, 0)),
            pl.BlockSpec((1, EMBED_DIM), lambda i: (0, 0)),
            pl.BlockSpec((EMBED_DIM, EMBED_DIM), lambda i: (0, 0)),
            pl.BlockSpec((1, EMBED_DIM), lambda i: (0, 0)),
        ],
        out_specs=pl.BlockSpec((blk, EMBED_DIM), lambda i: (i, 0)),
        out_shape=jax.ShapeDtypeStruct((BATCH, EMBED_DIM), jnp.float32),
        compiler_params=pltpu.CompilerParams(
            dimension_semantics=("parallel",),
        ),
    )(emb, W1, b1.reshape(1, EMBED_DIM), W2, b2.reshape(1, EMBED_DIM))


def kernel(class_index, table, W1, b1, W2, b2):
    idx3 = class_index.astype(jnp.int32).reshape(NW, N_CHUNKS, CHUNK)
    emb = _sc_gather(table, idx3)
    return _tc_mlp(emb, W1, b1, W2, b2)
